# baseline (speedup 1.0000x reference)
import functools

import jax
import jax.numpy as jnp
from jax import lax
from jax.experimental import pallas as pl
from jax.experimental.pallas import tpu as pltpu
from jax.experimental.pallas import tpu_sc as plsc

NUM_POSITIONS = 8192
EMBEDDING_DIM = 2048
TOTAL = 4 * 8192

NUM_WORKERS = 32
R = 16
NBLK = NUM_POSITIONS // R
NB = NBLK // NUM_WORKERS
C = 8
W = 256
WE = W - 16
PADDED = TOTAL + W
SW = 32
POS_BITS = 15


def _extract(vec_ref, j):
    base = pl.multiple_of((j // 16) * 16, 8)
    v = vec_ref[pl.ds(base, 16)]
    return jnp.sum(jnp.where(lax.iota(jnp.int32, 16) == j - base, v, 0))


def _emb_body(sidx_hbm, order_hbm, order2_hbm, starts_hbm, table_hbm, out_hbm,
              bufs, stage, sidxw, orderw, order2w, startsw,
              rsems, ssems, wsem):
    nc = plsc.get_sparse_core_info().num_cores
    wid = lax.axis_index("s") * nc + lax.axis_index("c")
    blk0 = wid * NB

    pltpu.sync_copy(starts_hbm.at[pl.ds(blk0, SW)], startsw)

    def read(b_loc, buf):
        return pltpu.make_async_copy(
            table_hbm.at[pl.ds((blk0 + b_loc) * R, R)], bufs.at[buf],
            rsems.at[buf]
        )

    def frag_write(j, wb, buf, row_base):
        src = _extract(sidxw, j - wb) - row_base
        dst = _extract(orderw, j - wb)
        pltpu.make_async_copy(
            bufs.at[buf, pl.ds(src, 1)], out_hbm.at[pl.ds(dst, 1)], wsem
        ).start()

    def frag_drain():
        pltpu.make_async_copy(
            table_hbm.at[pl.ds(0, 1)], bufs.at[0, pl.ds(0, 1)], wsem
        ).wait()

    def chunk_scatter(p, r):
        return pltpu.make_async_copy(
            stage.at[pl.ds(pl.multiple_of(p * C, 8), C)],
            out_hbm.at[order2w.at[r]],
            ssems.at[p],
        )

    def process_block(b_loc, buf, s_lo, s_hi, nch0):
        row_base = (blk0 + b_loc) * R

        def window(carry):
            j0, nch, nfrag = carry
            wb = pl.multiple_of((j0 // 64) * 64, 64)
            pltpu.sync_copy(sidx_hbm.at[pl.ds(wb, W)], sidxw)
            pltpu.sync_copy(order_hbm.at[pl.ds(wb, W)], orderw)
            pltpu.sync_copy(
                order2_hbm.at[pl.ds(pl.multiple_of(wb // C, 8), W // C)],
                order2w,
            )
            jend = jnp.minimum(s_hi, wb + WE)

            ja = jnp.minimum(jend, ((j0 + C - 1) // C) * C)
            lax.fori_loop(
                j0, ja, lambda j, _: (frag_write(j, wb, buf, row_base), 0)[1], 0
            )
            nfrag += ja - j0

            nfull = (jend - ja) // C

            def chunk(c2, nch):
                pos = ja + c2 * C
                p = lax.rem(nch, 2)

                @pl.when(nch >= 2)
                def _():
                    chunk_scatter(p, 0).wait()

                rel = pos - wb
                for e in range(C):
                    src = _extract(sidxw, rel + e) - row_base
                    srow = p * C + e

                    def cp(k, _):
                        off = pl.multiple_of(k * 128, 8)
                        for u in range(8):
                            stage[srow, pl.ds(off + u * 16, 16)] = (
                                bufs[buf, src, pl.ds(off + u * 16, 16)]
                            )
                        return 0

                    lax.fori_loop(0, EMBEDDING_DIM // 128, cp, 0)

                chunk_scatter(p, rel // C).start()
                return nch + 1

            nch = lax.fori_loop(0, nfull, chunk, nch)

            jr = ja + nfull * C
            lax.fori_loop(
                jr, jend, lambda j, _: (frag_write(j, wb, buf, row_base), 0)[1], 0
            )
            nfrag += jend - jr
            return jend, nch, nfrag

        return lax.while_loop(
            lambda c: c[0] < s_hi, window, (s_lo, nch0, jnp.int32(0))
        )

    read(0, 0).start()

    def body(b_loc, carry):
        frag_prev, nch = carry
        buf = lax.rem(b_loc, 2)
        s_lo = _extract(startsw, b_loc)
        s_hi = _extract(startsw, b_loc + 1)

        lax.fori_loop(0, frag_prev, lambda _, c: (frag_drain(), c)[1], 0)

        @pl.when(b_loc + 1 < NB)
        def _():
            read(b_loc + 1, 1 - buf).start()

        read(b_loc, buf).wait()
        _, nch, nfrag = process_block(b_loc, buf, s_lo, s_hi, nch)
        return nfrag, nch

    last_frag, nch = lax.fori_loop(0, NB, body, (jnp.int32(0), jnp.int32(0)))
    lax.fori_loop(0, last_frag, lambda _, c: (frag_drain(), c)[1], 0)

    @pl.when(nch >= 1)
    def _():
        chunk_scatter(lax.rem(nch - 1, 2), 0).wait()

    @pl.when(nch >= 2)
    def _():
        chunk_scatter(lax.rem(nch - 2, 2), 0).wait()


@functools.partial(
    pl.kernel,
    out_type=jax.ShapeDtypeStruct((TOTAL, EMBEDDING_DIM), jnp.float32),
    mesh=plsc.VectorSubcoreMesh(core_axis_name="c", subcore_axis_name="s"),
    compiler_params=pltpu.CompilerParams(needs_layout_passes=False),
    scratch_types=[
        pltpu.VMEM((2, R, EMBEDDING_DIM), jnp.float32),
        pltpu.VMEM((2 * C, EMBEDDING_DIM), jnp.float32),
        pltpu.VMEM((W,), jnp.int32),
        pltpu.VMEM((W,), jnp.int32),
        pltpu.VMEM((W // C, C), jnp.int32),
        pltpu.VMEM((SW,), jnp.int32),
        pltpu.SemaphoreType.DMA((2,)),
        pltpu.SemaphoreType.DMA((2,)),
        pltpu.SemaphoreType.DMA,
    ],
)
def _emb(sidx_hbm, order_hbm, order2_hbm, starts_hbm, table_hbm, out_hbm,
         bufs, stage, sidxw, orderw, order2w, startsw, rsems, ssems, wsem):
    _emb_body(sidx_hbm, order_hbm, order2_hbm, starts_hbm, table_hbm, out_hbm,
              bufs, stage, sidxw, orderw, order2w, startsw,
              rsems, ssems, wsem)


def kernel(positions, weight):
    flat = positions.reshape(-1).astype(jnp.int32)
    key = jnp.sort(flat * (1 << POS_BITS) + jnp.arange(TOTAL, dtype=jnp.int32))
    sidx = key >> POS_BITS
    order = key & ((1 << POS_BITS) - 1)
    starts = jnp.searchsorted(
        key,
        jnp.arange(NBLK + 1, dtype=jnp.int32) * (R << POS_BITS),
    ).astype(jnp.int32)
    sidx_p = jnp.pad(sidx, (0, PADDED - TOTAL))
    order_p = jnp.pad(order, (0, PADDED - TOTAL))
    starts_p = jnp.pad(starts, (0, NBLK + SW - starts.shape[0]),
                       constant_values=TOTAL)
    out = _emb(sidx_p, order_p, order_p.reshape(-1, C), starts_p, weight)
    return out.reshape(positions.shape + (weight.shape[1],))
